# baseline (speedup 1.0000x reference)
import numpy as np

import jax
import jax.numpy as jnp
from jax import lax
from jax.experimental import pallas as pl
from jax.experimental.pallas import tpu as pltpu
from jax.experimental.pallas import tpu_sc as plsc

N = 10000
NPAD = 10240
E = 320000
EP = 32 * 80 * 132
AW = 16
C = 80
K = EP // (32 * C)
STRIPE = NPAD // 16
BLK = 2048


def _stage_a_body(x_ref, w_ref, as_ref, ad_ref, h_ref, asw_ref, adw_ref,
                  ms_ref, md_ref):
    h = jnp.dot(x_ref[...], w_ref[...], preferred_element_type=jnp.float32)
    h_ref[...] = h.astype(jnp.bfloat16)
    a_s = jnp.dot(h, as_ref[...], preferred_element_type=jnp.float32)
    a_d = jnp.dot(h, ad_ref[...], preferred_element_type=jnp.float32)
    asw_ref[...] = a_s
    adw_ref[...] = a_d
    cur_s = jnp.broadcast_to(jnp.max(a_s, axis=0, keepdims=True), (8, AW))
    cur_d = jnp.broadcast_to(jnp.max(a_d, axis=0, keepdims=True), (8, AW))

    @pl.when(pl.program_id(0) == 0)
    def _():
        ms_ref[...] = cur_s
        md_ref[...] = cur_d

    @pl.when(pl.program_id(0) != 0)
    def _():
        ms_ref[...] = jnp.maximum(ms_ref[...], cur_s)
        md_ref[...] = jnp.maximum(md_ref[...], cur_d)


def _stage_a(xp, w, a_sm, a_dm, din, dout):
    grid = NPAD // BLK
    h, asw, adw, ms, md = pl.pallas_call(
        _stage_a_body,
        grid=(grid,),
        in_specs=[
            pl.BlockSpec((BLK, din), lambda i: (i, 0)),
            pl.BlockSpec((din, dout), lambda i: (0, 0)),
            pl.BlockSpec((dout, AW), lambda i: (0, 0)),
            pl.BlockSpec((dout, AW), lambda i: (0, 0)),
        ],
        out_specs=[
            pl.BlockSpec((BLK, dout), lambda i: (i, 0)),
            pl.BlockSpec((BLK, AW), lambda i: (i, 0)),
            pl.BlockSpec((BLK, AW), lambda i: (i, 0)),
            pl.BlockSpec((8, AW), lambda i: (0, 0)),
            pl.BlockSpec((8, AW), lambda i: (0, 0)),
        ],
        out_shape=[
            jax.ShapeDtypeStruct((NPAD, dout), jnp.bfloat16),
            jax.ShapeDtypeStruct((NPAD, AW), jnp.float32),
            jax.ShapeDtypeStruct((NPAD, AW), jnp.float32),
            jax.ShapeDtypeStruct((8, AW), jnp.float32),
            jax.ShapeDtypeStruct((8, AW), jnp.float32),
        ],
    )(xp, w, a_sm, a_dm)
    msum = jnp.max(ms, axis=0) + jnp.max(md, axis=0)
    g16 = jnp.maximum(msum, 0.2 * msum)
    return h, asw, adw, g16


def _lane_splat(vec, lane):
    idx = jnp.full((16, 1), lane, dtype=jnp.int32)
    return lax.gather(
        vec, idx,
        dimension_numbers=lax.GatherDimensionNumbers(
            offset_dims=(), collapsed_slice_dims=(0,), start_index_map=(0,)),
        slice_sizes=(1,),
        mode=lax.GatherScatterMode.PROMISE_IN_BOUNDS)


def _make_edge_kernel(d_feat, hid):
    mesh = plsc.VectorSubcoreMesh(core_axis_name="c", subcore_axis_name="s")
    kt = K + 2

    def body(src_hbm, dst_hbm, h_hbm, as_hbm, ad_hbm, g_hbm, zo_hbm, zd_hbm,
             out_hbm, den_hbm,
             src0, dst0, src1, dst1, src2, dst2,
             h0, m0, as0, ad0, ex0,
             h1, m1, as1, ad1, ex1,
             g_v, out_sh, den_sh,
             si0, si1, si2, sg0, sg1, ss0, ss1):
        cid = lax.axis_index("c")
        sid = lax.axis_index("s")
        srcs, dsts = [src0, src1, src2], [dst0, dst1, dst2]
        hs, msgs = [h0, h1], [m0, m1]
        ass, ads, exs = [as0, as1], [ad0, ad1], [ex0, ex1]
        sis, sgs, sss = [si0, si1, si2], [sg0, sg1], [ss0, ss1]

        pltpu.sync_copy(zo_hbm, out_sh.at[pl.ds(sid * STRIPE, STRIPE)])
        pltpu.sync_copy(zd_hbm, den_sh.at[pl.ds(sid * STRIPE, STRIPE)])
        pltpu.sync_copy(g_hbm, g_v)
        plsc.subcore_barrier()
        g = g_v[...]
        base = (cid * 16 + sid) * (kt * C)

        def issue_idx(k, i):
            eb = base + k * C
            pltpu.async_copy(src_hbm.at[pl.ds(eb, C)], srcs[i], sis[i])
            pltpu.async_copy(dst_hbm.at[pl.ds(eb, C)], dsts[i], sis[i])

        def wait_idx(i):
            pltpu.make_async_copy(src_hbm.at[pl.ds(0, C)], srcs[i], sis[i]).wait()
            pltpu.make_async_copy(dst_hbm.at[pl.ds(0, C)], dsts[i], sis[i]).wait()

        def issue_gathers(d, i):
            pltpu.async_copy(h_hbm.at[srcs[i]], hs[d], sgs[d])
            pltpu.async_copy(as_hbm.at[srcs[i]], ass[d], sgs[d])
            pltpu.async_copy(ad_hbm.at[dsts[i]], ads[d], sgs[d])

        def wait_gathers(d, i):
            pltpu.make_async_copy(h_hbm.at[srcs[i]], hs[d], sgs[d]).wait()
            pltpu.make_async_copy(as_hbm.at[srcs[i]], ass[d], sgs[d]).wait()
            pltpu.make_async_copy(ad_hbm.at[dsts[i]], ads[d], sgs[d]).wait()

        def issue_scatters(d, i):
            pltpu.async_copy(msgs[d], out_sh.at[dsts[i]], sss[d], add=True)
            pltpu.async_copy(exs[d], den_sh.at[dsts[i]], sss[d], add=True)

        def wait_scatters(d, i):
            pltpu.make_async_copy(msgs[d], out_sh.at[dsts[i]], sss[d]).wait()
            pltpu.make_async_copy(exs[d], den_sh.at[dsts[i]], sss[d]).wait()

        def compute(d):
            hv, mv, av, dv, ev = hs[d], msgs[d], ass[d], ads[d], exs[d]

            @plsc.parallel_loop(0, C, 1, unroll=8)
            def _(e):
                a = av[e] + dv[e]
                ve = jnp.exp(jnp.maximum(a, 0.2 * a) - g)
                ev[e] = ve
                for v2 in range(d_feat // 32):
                    hb = hv[e, pl.ds(v2 * 32, 32)]
                    pa, pb = plsc.unpack(hb, format=plsc.PackFormat.INTERLEAVED)
                    ca = _lane_splat(ve, (v2 * 32) // hid)
                    cb = _lane_splat(ve, (v2 * 32 + 16) // hid)
                    mv[e, pl.ds(v2 * 32, 16)] = pa * ca
                    mv[e, pl.ds(v2 * 32 + 16, 16)] = pb * cb

        issue_idx(0, 0)
        issue_idx(1, 1)
        wait_idx(0)
        issue_gathers(0, 0)

        def super_body(k0, carry):
            for b in range(6):
                k = 6 * k0 + b
                d, d1 = b % 2, (b + 1) % 2
                i0, i1, i2 = b % 3, (b + 1) % 3, (b + 2) % 3
                wait_idx(i1)
                if b == 0:
                    @pl.when(k0 > 0)
                    def _():
                        wait_scatters(d1, i2)
                else:
                    wait_scatters(d1, i2)
                issue_gathers(d1, i1)
                issue_idx(k + 2, i2)
                wait_gathers(d, i0)
                compute(d)
                issue_scatters(d, i0)
            return carry

        lax.fori_loop(0, K // 6, super_body, 0)
        wait_scatters((K - 1) % 2, (K - 1) % 3)
        wait_gathers(K % 2, K % 3)
        wait_idx((K + 1) % 3)
        plsc.subcore_barrier()
        r0 = sid * STRIPE
        pltpu.sync_copy(out_sh.at[pl.ds(r0, STRIPE)],
                        out_hbm.at[cid, pl.ds(r0, STRIPE)])
        pltpu.sync_copy(den_sh.at[pl.ds(r0, STRIPE)],
                        den_hbm.at[cid, pl.ds(r0, STRIPE)])

    dslot = [
        pltpu.VMEM((C, d_feat), jnp.bfloat16),
        pltpu.VMEM((C, d_feat), jnp.float32),
        pltpu.VMEM((C, AW), jnp.float32),
        pltpu.VMEM((C, AW), jnp.float32),
        pltpu.VMEM((C, AW), jnp.float32),
    ]
    islot = [pltpu.VMEM((C,), jnp.int32), pltpu.VMEM((C,), jnp.int32)]
    return pl.kernel(
        body,
        mesh=mesh,
        compiler_params=pltpu.CompilerParams(use_tc_tiling_on_sc=False,
                                             needs_layout_passes=False),
        out_type=[
            jax.ShapeDtypeStruct((2, NPAD, d_feat), jnp.float32),
            jax.ShapeDtypeStruct((2, NPAD, AW), jnp.float32),
        ],
        scratch_types=(islot * 3) + (dslot * 2) + [
            pltpu.VMEM((16,), jnp.float32),
            pltpu.VMEM_SHARED((NPAD, d_feat), jnp.float32),
            pltpu.VMEM_SHARED((NPAD, AW), jnp.float32),
        ] + [pltpu.SemaphoreType.DMA] * 7,
    )


_EDGE128 = _make_edge_kernel(128, 16)
_EDGE64 = _make_edge_kernel(64, 64)


def _combine_elu_body(p_ref, den_ref, r_ref, b_ref, o_ref):
    num = p_ref[0] + p_ref[1]
    den = jnp.dot(den_ref[0] + den_ref[1], r_ref[...],
                  preferred_element_type=jnp.float32)
    o = num / den + b_ref[0:1, :]
    o_ref[...] = jnp.where(o > 0, o, jnp.exp(jnp.minimum(o, 0.0)) - 1.0)


def _combine_lsm_body(p_ref, den_ref, r_ref, b_ref, o_ref):
    num = p_ref[0] + p_ref[1]
    den = jnp.dot(den_ref[0] + den_ref[1], r_ref[...],
                  preferred_element_type=jnp.float32)
    o = num / den + b_ref[0:1, :]
    m = jnp.max(o, axis=1, keepdims=True)
    ls = o - m
    o_ref[...] = ls - jnp.log(jnp.sum(jnp.exp(ls), axis=1, keepdims=True))


def _fused_body(p_ref, den_ref, r_ref, b_ref, w_ref, as_ref, ad_ref,
                h_ref, asw_ref, adw_ref, ms_ref, md_ref):
    num = p_ref[0] + p_ref[1]
    den = jnp.dot(den_ref[0] + den_ref[1], r_ref[...],
                  preferred_element_type=jnp.float32)
    o = num / den + b_ref[0:1, :]
    x = jnp.where(o > 0, o, jnp.exp(jnp.minimum(o, 0.0)) - 1.0)
    h = jnp.dot(x, w_ref[...], preferred_element_type=jnp.float32)
    h_ref[...] = h.astype(jnp.bfloat16)
    a_s = jnp.dot(h, as_ref[...], preferred_element_type=jnp.float32)
    a_d = jnp.dot(h, ad_ref[...], preferred_element_type=jnp.float32)
    asw_ref[...] = a_s
    adw_ref[...] = a_d
    cur_s = jnp.broadcast_to(jnp.max(a_s, axis=0, keepdims=True), (8, AW))
    cur_d = jnp.broadcast_to(jnp.max(a_d, axis=0, keepdims=True), (8, AW))

    @pl.when(pl.program_id(0) == 0)
    def _():
        ms_ref[...] = cur_s
        md_ref[...] = cur_d

    @pl.when(pl.program_id(0) != 0)
    def _():
        ms_ref[...] = jnp.maximum(ms_ref[...], cur_s)
        md_ref[...] = jnp.maximum(md_ref[...], cur_d)


def _fused_combine_a(p, den, r, b8, w, a_sm, a_dm, din, dout):
    grid = NPAD // BLK
    h, asw, adw, ms, md = pl.pallas_call(
        _fused_body,
        grid=(grid,),
        in_specs=[
            pl.BlockSpec((2, BLK, din), lambda i: (0, i, 0)),
            pl.BlockSpec((2, BLK, AW), lambda i: (0, i, 0)),
            pl.BlockSpec((AW, din), lambda i: (0, 0)),
            pl.BlockSpec((8, din), lambda i: (0, 0)),
            pl.BlockSpec((din, dout), lambda i: (0, 0)),
            pl.BlockSpec((dout, AW), lambda i: (0, 0)),
            pl.BlockSpec((dout, AW), lambda i: (0, 0)),
        ],
        out_specs=[
            pl.BlockSpec((BLK, dout), lambda i: (i, 0)),
            pl.BlockSpec((BLK, AW), lambda i: (i, 0)),
            pl.BlockSpec((BLK, AW), lambda i: (i, 0)),
            pl.BlockSpec((8, AW), lambda i: (0, 0)),
            pl.BlockSpec((8, AW), lambda i: (0, 0)),
        ],
        out_shape=[
            jax.ShapeDtypeStruct((NPAD, dout), jnp.bfloat16),
            jax.ShapeDtypeStruct((NPAD, AW), jnp.float32),
            jax.ShapeDtypeStruct((NPAD, AW), jnp.float32),
            jax.ShapeDtypeStruct((8, AW), jnp.float32),
            jax.ShapeDtypeStruct((8, AW), jnp.float32),
        ],
    )(p, den, r, b8, w, a_sm, a_dm)
    msum = jnp.max(ms, axis=0) + jnp.max(md, axis=0)
    g16 = jnp.maximum(msum, 0.2 * msum)
    return h, asw, adw, g16


def _combine(body, p, den, r, b8, d_feat):
    grid = NPAD // BLK
    return pl.pallas_call(
        body,
        grid=(grid,),
        in_specs=[
            pl.BlockSpec((2, BLK, d_feat), lambda i: (0, i, 0)),
            pl.BlockSpec((2, BLK, AW), lambda i: (0, i, 0)),
            pl.BlockSpec((AW, d_feat), lambda i: (0, 0)),
            pl.BlockSpec((8, d_feat), lambda i: (0, 0)),
        ],
        out_specs=pl.BlockSpec((BLK, d_feat), lambda i: (i, 0)),
        out_shape=jax.ShapeDtypeStruct((NPAD, d_feat), jnp.float32),
    )(p, den, r, b8)


def _interleave_perm(d):
    p = [0] * d
    for blk in range(d // 32):
        for j in range(16):
            p[blk * 32 + 2 * j] = blk * 32 + j
            p[blk * 32 + 2 * j + 1] = blk * 32 + 16 + j
    return np.array(p)


_P128 = _interleave_perm(128)
_P64 = _interleave_perm(64)


def _head_mats(a_s, a_d, heads, hid, d_feat, perm):
    eye = jnp.eye(heads, dtype=jnp.float32)
    a_sm = (a_s[:, :, None] * eye[:, None, :]).reshape(heads * hid, heads)
    a_dm = (a_d[:, :, None] * eye[:, None, :]).reshape(heads * hid, heads)
    a_sm = jnp.pad(a_sm, ((0, d_feat - heads * hid), (0, AW - heads)))[perm]
    a_dm = jnp.pad(a_dm, ((0, d_feat - heads * hid), (0, AW - heads)))[perm]
    rmat = jnp.pad(jnp.repeat(jnp.eye(heads, dtype=jnp.float32), hid, axis=1),
                   ((0, AW - heads), (0, 0)))
    return a_sm, a_dm, rmat


def kernel(x, edge_index, W0, a_s0, a_d0, b0, W1, a_s1, a_d1, b1,
           W2, a_s2, a_d2, b2):
    f32 = jnp.float32
    loop = jnp.arange(N, dtype=jnp.int32)
    pad_n = EP - (E + N)
    pad_idx = N + (jnp.arange(pad_n, dtype=jnp.int32) % (NPAD - N))

    def _tile_layout(v):
        v = v.reshape(32, K * C)
        v = jnp.pad(v, ((0, 0), (0, 2 * C)), constant_values=N)
        return v.reshape(-1)

    src = _tile_layout(jnp.concatenate([edge_index[0].astype(jnp.int32), loop, pad_idx]))
    dst = _tile_layout(jnp.concatenate([edge_index[1].astype(jnp.int32), loop, pad_idx]))

    xp = jnp.pad(x, ((0, NPAD - N), (0, 0)))
    zo128 = jnp.zeros((STRIPE, 128), f32)
    zo64 = jnp.zeros((STRIPE, 64), f32)
    zd = jnp.zeros((STRIPE, AW), f32)

    a_sm, a_dm, rmat0 = _head_mats(a_s0, a_d0, 8, 16, 128, _P128)
    h, asw, adw, g16 = _stage_a(xp, W0[:, _P128], a_sm, a_dm, 128, 128)
    p, den = _EDGE128(src, dst, h, asw, adw, g16, zo128, zd)

    a_sm, a_dm, rmat1 = _head_mats(a_s1, a_d1, 8, 16, 128, _P128)
    h, asw, adw, g16 = _fused_combine_a(p, den, rmat0,
                                        jnp.broadcast_to(b0, (8, 128)),
                                        W1[:, _P128], a_sm, a_dm, 128, 128)
    p, den = _EDGE128(src, dst, h, asw, adw, g16, zo128, zd)

    a_sm, a_dm, rmat2 = _head_mats(a_s2, a_d2, 1, 64, 64, _P64)
    h, asw, adw, g16 = _fused_combine_a(p, den, rmat1,
                                        jnp.broadcast_to(b1, (8, 128)),
                                        W2[:, _P64], a_sm, a_dm, 128, 64)
    p, den = _EDGE64(src, dst, h, asw, adw, g16, zo64, zd)
    out = _combine(_combine_lsm_body, p, den, rmat2,
                   jnp.broadcast_to(b2, (8, 64)), 64)
    return out[:N]


# speedup vs baseline: 1.0172x; 1.0172x over previous
import numpy as np

import jax
import jax.numpy as jnp
from jax import lax
from jax.experimental import pallas as pl
from jax.experimental.pallas import tpu as pltpu
from jax.experimental.pallas import tpu_sc as plsc

N = 10000
NPAD = 10240
E = 320000
EP = 32 * 128 * 81
AW = 16
C = 72
K = EP // (32 * C)
STRIPE = NPAD // 16
BLK = 2048


def _stage_a_body(x_ref, w_ref, as_ref, ad_ref, h_ref, asw_ref, adw_ref,
                  ms_ref, md_ref):
    h = jnp.dot(x_ref[...], w_ref[...], preferred_element_type=jnp.float32)
    h_ref[...] = h.astype(jnp.bfloat16)
    a_s = jnp.dot(h, as_ref[...], preferred_element_type=jnp.float32)
    a_d = jnp.dot(h, ad_ref[...], preferred_element_type=jnp.float32)
    asw_ref[...] = a_s
    adw_ref[...] = a_d
    cur_s = jnp.broadcast_to(jnp.max(a_s, axis=0, keepdims=True), (8, AW))
    cur_d = jnp.broadcast_to(jnp.max(a_d, axis=0, keepdims=True), (8, AW))

    @pl.when(pl.program_id(0) == 0)
    def _():
        ms_ref[...] = cur_s
        md_ref[...] = cur_d

    @pl.when(pl.program_id(0) != 0)
    def _():
        ms_ref[...] = jnp.maximum(ms_ref[...], cur_s)
        md_ref[...] = jnp.maximum(md_ref[...], cur_d)


def _stage_a(xp, w, a_sm, a_dm, din, dout):
    grid = NPAD // BLK
    h, asw, adw, ms, md = pl.pallas_call(
        _stage_a_body,
        grid=(grid,),
        in_specs=[
            pl.BlockSpec((BLK, din), lambda i: (i, 0)),
            pl.BlockSpec((din, dout), lambda i: (0, 0)),
            pl.BlockSpec((dout, AW), lambda i: (0, 0)),
            pl.BlockSpec((dout, AW), lambda i: (0, 0)),
        ],
        out_specs=[
            pl.BlockSpec((BLK, dout), lambda i: (i, 0)),
            pl.BlockSpec((BLK, AW), lambda i: (i, 0)),
            pl.BlockSpec((BLK, AW), lambda i: (i, 0)),
            pl.BlockSpec((8, AW), lambda i: (0, 0)),
            pl.BlockSpec((8, AW), lambda i: (0, 0)),
        ],
        out_shape=[
            jax.ShapeDtypeStruct((NPAD, dout), jnp.bfloat16),
            jax.ShapeDtypeStruct((NPAD, AW), jnp.float32),
            jax.ShapeDtypeStruct((NPAD, AW), jnp.float32),
            jax.ShapeDtypeStruct((8, AW), jnp.float32),
            jax.ShapeDtypeStruct((8, AW), jnp.float32),
        ],
    )(xp, w, a_sm, a_dm)
    msum = jnp.max(ms, axis=0) + jnp.max(md, axis=0)
    g16 = jnp.maximum(msum, 0.2 * msum)
    return h, asw, adw, g16


def _lane_splat(vec, lane):
    idx = jnp.full((16, 1), lane, dtype=jnp.int32)
    return lax.gather(
        vec, idx,
        dimension_numbers=lax.GatherDimensionNumbers(
            offset_dims=(), collapsed_slice_dims=(0,), start_index_map=(0,)),
        slice_sizes=(1,),
        mode=lax.GatherScatterMode.PROMISE_IN_BOUNDS)


def _make_edge_kernel(d_feat, hid):
    mesh = plsc.VectorSubcoreMesh(core_axis_name="c", subcore_axis_name="s")
    kt = K + 2

    def body(src_hbm, dst_hbm, h_hbm, as_hbm, ad_hbm, g_hbm, zo_hbm, zd_hbm,
             out_hbm, den_hbm,
             src0, dst0, src1, dst1, src2, dst2,
             h0, m0, as0, ad0, ex0,
             h1, m1, as1, ad1, ex1,
             g_v, out_sh, den_sh,
             si0, si1, si2, sg0, sg1, ss0, ss1):
        cid = lax.axis_index("c")
        sid = lax.axis_index("s")
        srcs, dsts = [src0, src1, src2], [dst0, dst1, dst2]
        hs, msgs = [h0, h1], [m0, m1]
        ass, ads, exs = [as0, as1], [ad0, ad1], [ex0, ex1]
        sis, sgs, sss = [si0, si1, si2], [sg0, sg1], [ss0, ss1]

        pltpu.sync_copy(zo_hbm, out_sh.at[pl.ds(sid * STRIPE, STRIPE)])
        pltpu.sync_copy(zd_hbm, den_sh.at[pl.ds(sid * STRIPE, STRIPE)])
        pltpu.sync_copy(g_hbm, g_v)
        plsc.subcore_barrier()
        g = g_v[...]
        base = (cid * 16 + sid) * (kt * C)

        def issue_idx(k, i):
            eb = base + k * C
            pltpu.async_copy(src_hbm.at[pl.ds(eb, C)], srcs[i], sis[i])
            pltpu.async_copy(dst_hbm.at[pl.ds(eb, C)], dsts[i], sis[i])

        def wait_idx(i):
            pltpu.make_async_copy(src_hbm.at[pl.ds(0, C)], srcs[i], sis[i]).wait()
            pltpu.make_async_copy(dst_hbm.at[pl.ds(0, C)], dsts[i], sis[i]).wait()

        def issue_gathers(d, i):
            pltpu.async_copy(h_hbm.at[srcs[i]], hs[d], sgs[d])
            pltpu.async_copy(as_hbm.at[srcs[i]], ass[d], sgs[d])
            pltpu.async_copy(ad_hbm.at[dsts[i]], ads[d], sgs[d])

        def wait_gathers(d, i):
            pltpu.make_async_copy(h_hbm.at[srcs[i]], hs[d], sgs[d]).wait()
            pltpu.make_async_copy(as_hbm.at[srcs[i]], ass[d], sgs[d]).wait()
            pltpu.make_async_copy(ad_hbm.at[dsts[i]], ads[d], sgs[d]).wait()

        def issue_scatters(d, i):
            pltpu.async_copy(msgs[d], out_sh.at[dsts[i]], sss[d], add=True)
            pltpu.async_copy(exs[d], den_sh.at[dsts[i]], sss[d], add=True)

        def wait_scatters(d, i):
            pltpu.make_async_copy(msgs[d], out_sh.at[dsts[i]], sss[d]).wait()
            pltpu.make_async_copy(exs[d], den_sh.at[dsts[i]], sss[d]).wait()

        def compute(d):
            hv, mv, av, dv, ev = hs[d], msgs[d], ass[d], ads[d], exs[d]

            @plsc.parallel_loop(0, C, 1, unroll=6)
            def _(e):
                a = av[e] + dv[e]
                ve = jnp.exp(jnp.maximum(a, 0.2 * a) - g)
                ev[e] = ve
                for v2 in range(d_feat // 32):
                    hb = hv[e, pl.ds(v2 * 32, 32)]
                    pa, pb = plsc.unpack(hb, format=plsc.PackFormat.INTERLEAVED)
                    ca = _lane_splat(ve, (v2 * 32) // hid)
                    cb = _lane_splat(ve, (v2 * 32 + 16) // hid)
                    mv[e, pl.ds(v2 * 32, 16)] = pa * ca
                    mv[e, pl.ds(v2 * 32 + 16, 16)] = pb * cb

        issue_idx(0, 0)
        issue_idx(1, 1)
        wait_idx(0)
        issue_gathers(0, 0)

        def super_body(k0, carry):
            for b in range(6):
                k = 6 * k0 + b
                d, d1 = b % 2, (b + 1) % 2
                i0, i1, i2 = b % 3, (b + 1) % 3, (b + 2) % 3
                wait_idx(i1)
                if b == 0:
                    @pl.when(k0 > 0)
                    def _():
                        wait_scatters(d1, i2)
                else:
                    wait_scatters(d1, i2)
                issue_gathers(d1, i1)
                issue_idx(k + 2, i2)
                wait_gathers(d, i0)
                compute(d)
                issue_scatters(d, i0)
            return carry

        lax.fori_loop(0, K // 6, super_body, 0)
        wait_scatters((K - 1) % 2, (K - 1) % 3)
        wait_gathers(K % 2, K % 3)
        wait_idx((K + 1) % 3)
        plsc.subcore_barrier()
        r0 = sid * STRIPE
        pltpu.sync_copy(out_sh.at[pl.ds(r0, STRIPE)],
                        out_hbm.at[cid, pl.ds(r0, STRIPE)])
        pltpu.sync_copy(den_sh.at[pl.ds(r0, STRIPE)],
                        den_hbm.at[cid, pl.ds(r0, STRIPE)])

    dslot = [
        pltpu.VMEM((C, d_feat), jnp.bfloat16),
        pltpu.VMEM((C, d_feat), jnp.float32),
        pltpu.VMEM((C, AW), jnp.float32),
        pltpu.VMEM((C, AW), jnp.float32),
        pltpu.VMEM((C, AW), jnp.float32),
    ]
    islot = [pltpu.VMEM((C,), jnp.int32), pltpu.VMEM((C,), jnp.int32)]
    return pl.kernel(
        body,
        mesh=mesh,
        compiler_params=pltpu.CompilerParams(use_tc_tiling_on_sc=False,
                                             needs_layout_passes=False),
        out_type=[
            jax.ShapeDtypeStruct((2, NPAD, d_feat), jnp.float32),
            jax.ShapeDtypeStruct((2, NPAD, AW), jnp.float32),
        ],
        scratch_types=(islot * 3) + (dslot * 2) + [
            pltpu.VMEM((16,), jnp.float32),
            pltpu.VMEM_SHARED((NPAD, d_feat), jnp.float32),
            pltpu.VMEM_SHARED((NPAD, AW), jnp.float32),
        ] + [pltpu.SemaphoreType.DMA] * 7,
    )


_EDGE128 = _make_edge_kernel(128, 16)
_EDGE64 = _make_edge_kernel(64, 64)


def _combine_elu_body(p_ref, den_ref, r_ref, b_ref, o_ref):
    num = p_ref[0] + p_ref[1]
    den = jnp.dot(den_ref[0] + den_ref[1], r_ref[...],
                  preferred_element_type=jnp.float32)
    o = num / den + b_ref[0:1, :]
    o_ref[...] = jnp.where(o > 0, o, jnp.exp(jnp.minimum(o, 0.0)) - 1.0)


def _combine_lsm_body(p_ref, den_ref, r_ref, b_ref, o_ref):
    num = p_ref[0] + p_ref[1]
    den = jnp.dot(den_ref[0] + den_ref[1], r_ref[...],
                  preferred_element_type=jnp.float32)
    o = num / den + b_ref[0:1, :]
    m = jnp.max(o, axis=1, keepdims=True)
    ls = o - m
    o_ref[...] = ls - jnp.log(jnp.sum(jnp.exp(ls), axis=1, keepdims=True))


def _fused_body(p_ref, den_ref, r_ref, b_ref, w_ref, as_ref, ad_ref,
                h_ref, asw_ref, adw_ref, ms_ref, md_ref):
    num = p_ref[0] + p_ref[1]
    den = jnp.dot(den_ref[0] + den_ref[1], r_ref[...],
                  preferred_element_type=jnp.float32)
    o = num / den + b_ref[0:1, :]
    x = jnp.where(o > 0, o, jnp.exp(jnp.minimum(o, 0.0)) - 1.0)
    h = jnp.dot(x, w_ref[...], preferred_element_type=jnp.float32)
    h_ref[...] = h.astype(jnp.bfloat16)
    a_s = jnp.dot(h, as_ref[...], preferred_element_type=jnp.float32)
    a_d = jnp.dot(h, ad_ref[...], preferred_element_type=jnp.float32)
    asw_ref[...] = a_s
    adw_ref[...] = a_d
    cur_s = jnp.broadcast_to(jnp.max(a_s, axis=0, keepdims=True), (8, AW))
    cur_d = jnp.broadcast_to(jnp.max(a_d, axis=0, keepdims=True), (8, AW))

    @pl.when(pl.program_id(0) == 0)
    def _():
        ms_ref[...] = cur_s
        md_ref[...] = cur_d

    @pl.when(pl.program_id(0) != 0)
    def _():
        ms_ref[...] = jnp.maximum(ms_ref[...], cur_s)
        md_ref[...] = jnp.maximum(md_ref[...], cur_d)


def _fused_combine_a(p, den, r, b8, w, a_sm, a_dm, din, dout):
    grid = NPAD // BLK
    h, asw, adw, ms, md = pl.pallas_call(
        _fused_body,
        grid=(grid,),
        in_specs=[
            pl.BlockSpec((2, BLK, din), lambda i: (0, i, 0)),
            pl.BlockSpec((2, BLK, AW), lambda i: (0, i, 0)),
            pl.BlockSpec((AW, din), lambda i: (0, 0)),
            pl.BlockSpec((8, din), lambda i: (0, 0)),
            pl.BlockSpec((din, dout), lambda i: (0, 0)),
            pl.BlockSpec((dout, AW), lambda i: (0, 0)),
            pl.BlockSpec((dout, AW), lambda i: (0, 0)),
        ],
        out_specs=[
            pl.BlockSpec((BLK, dout), lambda i: (i, 0)),
            pl.BlockSpec((BLK, AW), lambda i: (i, 0)),
            pl.BlockSpec((BLK, AW), lambda i: (i, 0)),
            pl.BlockSpec((8, AW), lambda i: (0, 0)),
            pl.BlockSpec((8, AW), lambda i: (0, 0)),
        ],
        out_shape=[
            jax.ShapeDtypeStruct((NPAD, dout), jnp.bfloat16),
            jax.ShapeDtypeStruct((NPAD, AW), jnp.float32),
            jax.ShapeDtypeStruct((NPAD, AW), jnp.float32),
            jax.ShapeDtypeStruct((8, AW), jnp.float32),
            jax.ShapeDtypeStruct((8, AW), jnp.float32),
        ],
    )(p, den, r, b8, w, a_sm, a_dm)
    msum = jnp.max(ms, axis=0) + jnp.max(md, axis=0)
    g16 = jnp.maximum(msum, 0.2 * msum)
    return h, asw, adw, g16


def _combine(body, p, den, r, b8, d_feat):
    grid = NPAD // BLK
    return pl.pallas_call(
        body,
        grid=(grid,),
        in_specs=[
            pl.BlockSpec((2, BLK, d_feat), lambda i: (0, i, 0)),
            pl.BlockSpec((2, BLK, AW), lambda i: (0, i, 0)),
            pl.BlockSpec((AW, d_feat), lambda i: (0, 0)),
            pl.BlockSpec((8, d_feat), lambda i: (0, 0)),
        ],
        out_specs=pl.BlockSpec((BLK, d_feat), lambda i: (i, 0)),
        out_shape=jax.ShapeDtypeStruct((NPAD, d_feat), jnp.float32),
    )(p, den, r, b8)


def _interleave_perm(d):
    p = [0] * d
    for blk in range(d // 32):
        for j in range(16):
            p[blk * 32 + 2 * j] = blk * 32 + j
            p[blk * 32 + 2 * j + 1] = blk * 32 + 16 + j
    return np.array(p)


_P128 = _interleave_perm(128)
_P64 = _interleave_perm(64)


def _head_mats(a_s, a_d, heads, hid, d_feat, perm):
    eye = jnp.eye(heads, dtype=jnp.float32)
    a_sm = (a_s[:, :, None] * eye[:, None, :]).reshape(heads * hid, heads)
    a_dm = (a_d[:, :, None] * eye[:, None, :]).reshape(heads * hid, heads)
    a_sm = jnp.pad(a_sm, ((0, d_feat - heads * hid), (0, AW - heads)))[perm]
    a_dm = jnp.pad(a_dm, ((0, d_feat - heads * hid), (0, AW - heads)))[perm]
    rmat = jnp.pad(jnp.repeat(jnp.eye(heads, dtype=jnp.float32), hid, axis=1),
                   ((0, AW - heads), (0, 0)))
    return a_sm, a_dm, rmat


def kernel(x, edge_index, W0, a_s0, a_d0, b0, W1, a_s1, a_d1, b1,
           W2, a_s2, a_d2, b2):
    f32 = jnp.float32
    loop = jnp.arange(N, dtype=jnp.int32)
    pad_n = EP - (E + N)
    pad_idx = N + (jnp.arange(pad_n, dtype=jnp.int32) % (NPAD - N))

    def _tile_layout(v):
        v = v.reshape(32, K * C)
        v = jnp.pad(v, ((0, 0), (0, 2 * C)), constant_values=N)
        return v.reshape(-1)

    src = _tile_layout(jnp.concatenate([edge_index[0].astype(jnp.int32), loop, pad_idx]))
    dst = _tile_layout(jnp.concatenate([edge_index[1].astype(jnp.int32), loop, pad_idx]))

    xp = jnp.pad(x, ((0, NPAD - N), (0, 0)))
    zo128 = jnp.zeros((STRIPE, 128), f32)
    zo64 = jnp.zeros((STRIPE, 64), f32)
    zd = jnp.zeros((STRIPE, AW), f32)

    a_sm, a_dm, rmat0 = _head_mats(a_s0, a_d0, 8, 16, 128, _P128)
    h, asw, adw, g16 = _stage_a(xp, W0[:, _P128], a_sm, a_dm, 128, 128)
    p, den = _EDGE128(src, dst, h, asw, adw, g16, zo128, zd)

    a_sm, a_dm, rmat1 = _head_mats(a_s1, a_d1, 8, 16, 128, _P128)
    h, asw, adw, g16 = _fused_combine_a(p, den, rmat0,
                                        jnp.broadcast_to(b0, (8, 128)),
                                        W1[:, _P128], a_sm, a_dm, 128, 128)
    p, den = _EDGE128(src, dst, h, asw, adw, g16, zo128, zd)

    a_sm, a_dm, rmat2 = _head_mats(a_s2, a_d2, 1, 64, 64, _P64)
    h, asw, adw, g16 = _fused_combine_a(p, den, rmat1,
                                        jnp.broadcast_to(b1, (8, 128)),
                                        W2[:, _P64], a_sm, a_dm, 128, 64)
    p, den = _EDGE64(src, dst, h, asw, adw, g16, zo64, zd)
    out = _combine(_combine_lsm_body, p, den, rmat2,
                   jnp.broadcast_to(b2, (8, 64)), 64)
    return out[:N]
